# baseline (speedup 1.0000x reference)
import functools
import jax
import jax.numpy as jnp
from jax.experimental import pallas as pl

N = 50000
E = 800000
NG = 8

EBLK = 1600
NBLK = 5000


def _ln(h, g, b):
    m = jnp.mean(h, axis=-1, keepdims=True)
    v = jnp.mean((h - m) * (h - m), axis=-1, keepdims=True)
    return (h - m) * jax.lax.rsqrt(v + 1e-5) * g + b


def _edge_kernel(edge_attr_ref, rel_ref, bsrc_ref,
                 w1a_ref, w1b_ref, w1l_ref, b1_ref, g1_ref, bb1_ref,
                 w2_ref, b2_ref, g2_ref, bb2_ref,
                 w3_ref, b3_ref, g3_ref, bb3_ref,
                 e2n_w_ref, e2n_b_ref,
                 p_ref, dist_ref, eg_sum_ref, eg_cnt_ref):
    i = pl.program_id(0)
    rel = rel_ref[...]
    d2 = jnp.sum(rel * rel, axis=-1, keepdims=True)
    dist = jnp.sqrt(d2)
    inv = 1.0 / (dist + 1e-8)
    logd = jnp.log(dist + 1e-6)

    h = (jnp.dot(edge_attr_ref[...], w1a_ref[...],
                 preferred_element_type=jnp.float32)
         + jnp.dot(rel, w1b_ref[...], preferred_element_type=jnp.float32) * inv
         + logd * w1l_ref[...]
         + b1_ref[...])
    h = jax.nn.relu(h)
    h = _ln(h, g1_ref[...], bb1_ref[...])
    h = jax.nn.relu(jnp.dot(h, w2_ref[...], preferred_element_type=jnp.float32)
                    + b2_ref[...])
    h = _ln(h, g2_ref[...], bb2_ref[...])
    h = jax.nn.relu(jnp.dot(h, w3_ref[...], preferred_element_type=jnp.float32)
                    + b3_ref[...])
    emb = _ln(h, g3_ref[...], bb3_ref[...])

    p = jax.nn.relu(jnp.dot(emb, e2n_w_ref[...],
                            preferred_element_type=jnp.float32)
                    + e2n_b_ref[...])
    p_ref[...] = p
    dist_ref[...] = dist

    onehot = (bsrc_ref[...] == jax.lax.broadcasted_iota(jnp.int32, (1, NG), 1)
              ).astype(jnp.float32)
    eg_blk = jax.lax.dot_general(onehot, emb, (((0,), (0,)), ((), ())),
                                 preferred_element_type=jnp.float32)
    cnt_blk = jax.lax.dot_general(onehot, jnp.ones((EBLK, 1), jnp.float32),
                                  (((0,), (0,)), ((), ())),
                                  preferred_element_type=jnp.float32)

    @pl.when(i == 0)
    def _():
        eg_sum_ref[...] = jnp.zeros_like(eg_sum_ref)
        eg_cnt_ref[...] = jnp.zeros_like(eg_cnt_ref)

    eg_sum_ref[...] += eg_blk
    eg_cnt_ref[...] += cnt_blk


def _nodeA_kernel(ss_ref, cs_ref, sd_ref, cd_ref, wd_ref, g1w_ref,
                  xn_ref, hw_ref, dinv_ref, invdeg_ref):
    cs = jnp.maximum(cs_ref[...], 1.0)
    cd = jnp.maximum(cd_ref[...], 1.0)
    xn = 0.5 * (ss_ref[...] / cs + sd_ref[...] / cd)
    deg = wd_ref[...] + 1.0
    dinv_ref[...] = jax.lax.rsqrt(deg)
    invdeg_ref[...] = 1.0 / deg
    xn_ref[...] = xn
    hw_ref[...] = jnp.dot(xn, g1w_ref[...], preferred_element_type=jnp.float32)


def _nodeB_kernel(agg_ref, hw_ref, invdeg_ref, xn_ref,
                  g1b_ref, n1g_ref, n1b_ref, g2w_ref,
                  x1_ref, hw2_ref):
    pre = agg_ref[...] + hw_ref[...] * invdeg_ref[...] + g1b_ref[...]
    t = pre + xn_ref[...]
    z = _ln(t, n1g_ref[...], n1b_ref[...])
    x1 = 0.5 * z * (1.0 + jax.lax.erf(z * 0.7071067811865476))
    x1_ref[...] = x1
    hw2_ref[...] = jnp.dot(x1, g2w_ref[...], preferred_element_type=jnp.float32)


def _nodeC_kernel(agg_ref, hw2_ref, invdeg_ref, x1_ref,
                  g2b_ref, n2g_ref, n2b_ref, batch_ref,
                  ng_sum_ref, ng_cnt_ref):
    i = pl.program_id(0)
    pre = agg_ref[...] + hw2_ref[...] * invdeg_ref[...] + g2b_ref[...]
    xn2 = _ln(pre + x1_ref[...], n2g_ref[...], n2b_ref[...])
    onehot = (batch_ref[...] == jax.lax.broadcasted_iota(jnp.int32, (1, NG), 1)
              ).astype(jnp.float32)
    ng_blk = jax.lax.dot_general(onehot, xn2, (((0,), (0,)), ((), ())),
                                 preferred_element_type=jnp.float32)
    cnt_blk = jax.lax.dot_general(onehot, jnp.ones((NBLK, 1), jnp.float32),
                                  (((0,), (0,)), ((), ())),
                                  preferred_element_type=jnp.float32)

    @pl.when(i == 0)
    def _():
        ng_sum_ref[...] = jnp.zeros_like(ng_sum_ref)
        ng_cnt_ref[...] = jnp.zeros_like(ng_cnt_ref)

    ng_sum_ref[...] += ng_blk
    ng_cnt_ref[...] += cnt_blk


def _head_kernel(ng_sum_ref, ng_cnt_ref, eg_sum_ref, eg_cnt_ref,
                 w1a_ref, w1b_ref, b1_ref, w2_ref, b2_ref, out_ref):
    node_graph = ng_sum_ref[...] / jnp.maximum(ng_cnt_ref[...], 1.0)
    edge_graph = eg_sum_ref[...] / jnp.maximum(eg_cnt_ref[...], 1.0)
    g = jax.nn.relu(
        jnp.dot(node_graph, w1a_ref[...], preferred_element_type=jnp.float32)
        + jnp.dot(edge_graph, w1b_ref[...], preferred_element_type=jnp.float32)
        + b1_ref[...])
    out_ref[...] = (jnp.dot(g, w2_ref[...], preferred_element_type=jnp.float32)
                    + b2_ref[...])


def _full(shape):
    return pl.BlockSpec(shape, lambda *args: tuple(0 for _ in shape))


def kernel(x, edge_attr, array, edge_index, batch, ee_w1, ee_b1, ee_g1, ee_bb1, ee_w2, ee_b2, ee_g2, ee_bb2, ee_w3, ee_b3, ee_g3, ee_bb3, e2n_w, e2n_b, g1_w, g1_b, g2_w, g2_b, n1_g, n1_b, n2_g, n2_b, h_w1, h_b1, h_w2, h_b2):
    src = edge_index[0]
    dst = edge_index[1]

    rel3 = array[src] - array[dst]
    rel = jnp.pad(rel3, ((0, 0), (0, 1)))
    bsrc = batch[src][:, None]

    w1a = ee_w1[:16]
    w1b = jnp.pad(ee_w1[16:19], ((0, 1), (0, 0)))
    w1l = ee_w1[19:20]

    ngrid = E // EBLK
    p, dist, eg_sum, eg_cnt = pl.pallas_call(
        _edge_kernel,
        grid=(ngrid,),
        in_specs=[
            pl.BlockSpec((EBLK, 16), lambda i: (i, 0)),
            pl.BlockSpec((EBLK, 4), lambda i: (i, 0)),
            pl.BlockSpec((EBLK, 1), lambda i: (i, 0)),
            _full((16, 128)), _full((4, 128)), _full((1, 128)),
            _full((1, 128)), _full((1, 128)), _full((1, 128)),
            _full((128, 128)), _full((1, 128)), _full((1, 128)), _full((1, 128)),
            _full((128, 128)), _full((1, 128)), _full((1, 128)), _full((1, 128)),
            _full((128, 128)), _full((1, 128)),
        ],
        out_specs=[
            pl.BlockSpec((EBLK, 128), lambda i: (i, 0)),
            pl.BlockSpec((EBLK, 1), lambda i: (i, 0)),
            _full((NG, 128)),
            _full((NG, 1)),
        ],
        out_shape=[
            jax.ShapeDtypeStruct((E, 128), jnp.float32),
            jax.ShapeDtypeStruct((E, 1), jnp.float32),
            jax.ShapeDtypeStruct((NG, 128), jnp.float32),
            jax.ShapeDtypeStruct((NG, 1), jnp.float32),
        ],
    )(edge_attr, rel, bsrc,
      w1a, w1b, w1l, ee_b1[None], ee_g1[None], ee_bb1[None],
      ee_w2, ee_b2[None], ee_g2[None], ee_bb2[None],
      ee_w3, ee_b3[None], ee_g3[None], ee_bb3[None],
      e2n_w, e2n_b[None])

    distf = dist[:, 0]

    ones = jnp.ones((E,), jnp.float32)
    cnt_src = jnp.zeros((N, 1), jnp.float32).at[src, 0].add(ones)
    cnt_dst = jnp.zeros((N, 1), jnp.float32).at[dst, 0].add(ones)
    sum_src = jnp.zeros((N, 128), jnp.float32).at[src].add(p)
    sum_dst = jnp.zeros((N, 128), jnp.float32).at[dst].add(p)
    wsum_dst = jnp.zeros((N, 1), jnp.float32).at[dst, 0].add(distf)

    ngrid_n = N // NBLK
    nspec = pl.BlockSpec((NBLK, 128), lambda i: (i, 0))
    cspec = pl.BlockSpec((NBLK, 1), lambda i: (i, 0))

    xn, hw, dinv, invdeg = pl.pallas_call(
        _nodeA_kernel,
        grid=(ngrid_n,),
        in_specs=[nspec, cspec, nspec, cspec, cspec, _full((128, 128))],
        out_specs=[nspec, nspec, cspec, cspec],
        out_shape=[
            jax.ShapeDtypeStruct((N, 128), jnp.float32),
            jax.ShapeDtypeStruct((N, 128), jnp.float32),
            jax.ShapeDtypeStruct((N, 1), jnp.float32),
            jax.ShapeDtypeStruct((N, 1), jnp.float32),
        ],
    )(sum_src, cnt_src, sum_dst, cnt_dst, wsum_dst, g1_w)

    dinvf = dinv[:, 0]
    norm = dinvf[src] * distf * dinvf[dst]

    agg1 = jnp.zeros((N, 128), jnp.float32).at[dst].add(norm[:, None] * hw[src])

    x1, hw2 = pl.pallas_call(
        _nodeB_kernel,
        grid=(ngrid_n,),
        in_specs=[nspec, nspec, cspec, nspec,
                  _full((1, 128)), _full((1, 128)), _full((1, 128)),
                  _full((128, 128))],
        out_specs=[nspec, nspec],
        out_shape=[
            jax.ShapeDtypeStruct((N, 128), jnp.float32),
            jax.ShapeDtypeStruct((N, 128), jnp.float32),
        ],
    )(agg1, hw, invdeg, xn, g1_b[None], n1_g[None], n1_b[None], g2_w)

    agg2 = jnp.zeros((N, 128), jnp.float32).at[dst].add(norm[:, None] * hw2[src])

    ng_sum, ng_cnt = pl.pallas_call(
        _nodeC_kernel,
        grid=(ngrid_n,),
        in_specs=[nspec, nspec, cspec, nspec,
                  _full((1, 128)), _full((1, 128)), _full((1, 128)),
                  pl.BlockSpec((NBLK, 1), lambda i: (i, 0))],
        out_specs=[_full((NG, 128)), _full((NG, 1))],
        out_shape=[
            jax.ShapeDtypeStruct((NG, 128), jnp.float32),
            jax.ShapeDtypeStruct((NG, 1), jnp.float32),
        ],
    )(agg2, hw2, invdeg, x1, g2_b[None], n2_g[None], n2_b[None],
      batch[:, None])

    w2p = jnp.pad(h_w2, ((0, 0), (0, 126)))
    b2p = jnp.pad(h_b2, (0, 126))[None]
    out128 = pl.pallas_call(
        _head_kernel,
        in_specs=[_full((NG, 128)), _full((NG, 1)),
                  _full((NG, 128)), _full((NG, 1)),
                  _full((128, 128)), _full((128, 128)), _full((1, 128)),
                  _full((128, 128)), _full((1, 128))],
        out_specs=_full((NG, 128)),
        out_shape=jax.ShapeDtypeStruct((NG, 128), jnp.float32),
    )(ng_sum, ng_cnt, eg_sum, eg_cnt,
      h_w1[:128], h_w1[128:], h_b1[None], w2p, b2p)

    return out128[:, :2]


# speedup vs baseline: 1.0546x; 1.0546x over previous
import functools
import jax
import jax.numpy as jnp
from jax import lax
from jax.experimental import pallas as pl
from jax.experimental.pallas import tpu as pltpu
from jax.experimental.pallas import tpu_sc as plsc

N = 50000
E = 800000
NG = 8

EBLK = 1600
NBLK = 5000

SCW = 64
NWIN = E // SCW


def _sc_u_body(h_hbm, src_hbm, scl_hbm, out_hbm, scl_v, sidx, rows, sem):
    c = lax.axis_index("c")
    s = lax.axis_index("s")
    wid = s * 2 + c

    def win(k, _):
        widx = wid + 32 * k

        @pl.when(widx < NWIN)
        def _():
            wb = widx * SCW
            pltpu.sync_copy(scl_hbm.at[pl.ds(wb, SCW)], scl_v)
            pltpu.sync_copy(src_hbm.at[pl.ds(wb, SCW)], sidx)
            pltpu.async_copy(h_hbm.at[sidx], rows, sem).wait()

            def scale(g, _):
                base = g * 16
                nv = scl_v[pl.ds(base, 16)]
                for t in range(16):
                    r = base + t
                    for q in range(8):
                        sl = pl.ds(q * 16, 16)
                        rows[r, sl] = rows[r, sl] * nv[t]
                return 0

            lax.fori_loop(0, SCW // 16, scale, 0)
            pltpu.sync_copy(rows, out_hbm.at[pl.ds(wb, SCW)])
        return 0

    lax.fori_loop(0, (NWIN + 31) // 32, win, 0)


def _sc_u(h, src, scl):
    return pl.kernel(
        _sc_u_body,
        out_type=jax.ShapeDtypeStruct((E, 128), jnp.float32),
        mesh=plsc.VectorSubcoreMesh(core_axis_name="c", subcore_axis_name="s"),
        scratch_types=[
            pltpu.VMEM((SCW,), jnp.float32),
            pltpu.VMEM((SCW,), jnp.int32),
            pltpu.VMEM((SCW, 128), jnp.float32),
            pltpu.SemaphoreType.DMA,
        ],
    )(h, src, scl)


def _ln(h, g, b):
    m = jnp.mean(h, axis=-1, keepdims=True)
    v = jnp.mean((h - m) * (h - m), axis=-1, keepdims=True)
    return (h - m) * jax.lax.rsqrt(v + 1e-5) * g + b


def _edge_kernel(edge_attr_ref, rel_ref, bsrc_ref,
                 w1a_ref, w1b_ref, w1l_ref, b1_ref, g1_ref, bb1_ref,
                 w2_ref, b2_ref, g2_ref, bb2_ref,
                 w3_ref, b3_ref, g3_ref, bb3_ref,
                 e2n_w_ref, e2n_b_ref,
                 p_ref, dist_ref, eg_sum_ref, eg_cnt_ref):
    i = pl.program_id(0)
    rel = rel_ref[...]
    d2 = jnp.sum(rel * rel, axis=-1, keepdims=True)
    dist = jnp.sqrt(d2)
    inv = 1.0 / (dist + 1e-8)
    logd = jnp.log(dist + 1e-6)

    h = (jnp.dot(edge_attr_ref[...], w1a_ref[...],
                 preferred_element_type=jnp.float32)
         + jnp.dot(rel, w1b_ref[...], preferred_element_type=jnp.float32) * inv
         + logd * w1l_ref[...]
         + b1_ref[...])
    h = jax.nn.relu(h)
    h = _ln(h, g1_ref[...], bb1_ref[...])
    h = jax.nn.relu(jnp.dot(h, w2_ref[...], preferred_element_type=jnp.float32)
                    + b2_ref[...])
    h = _ln(h, g2_ref[...], bb2_ref[...])
    h = jax.nn.relu(jnp.dot(h, w3_ref[...], preferred_element_type=jnp.float32)
                    + b3_ref[...])
    emb = _ln(h, g3_ref[...], bb3_ref[...])

    p = jax.nn.relu(jnp.dot(emb, e2n_w_ref[...],
                            preferred_element_type=jnp.float32)
                    + e2n_b_ref[...])
    p_ref[...] = p
    dist_ref[...] = dist

    onehot = (bsrc_ref[...] == jax.lax.broadcasted_iota(jnp.int32, (1, NG), 1)
              ).astype(jnp.float32)
    eg_blk = jax.lax.dot_general(onehot, emb, (((0,), (0,)), ((), ())),
                                 preferred_element_type=jnp.float32)
    cnt_blk = jax.lax.dot_general(onehot, jnp.ones((EBLK, 1), jnp.float32),
                                  (((0,), (0,)), ((), ())),
                                  preferred_element_type=jnp.float32)

    @pl.when(i == 0)
    def _():
        eg_sum_ref[...] = jnp.zeros_like(eg_sum_ref)
        eg_cnt_ref[...] = jnp.zeros_like(eg_cnt_ref)

    eg_sum_ref[...] += eg_blk
    eg_cnt_ref[...] += cnt_blk


def _nodeA_kernel(ss_ref, cs_ref, sd_ref, cd_ref, wd_ref, g1w_ref,
                  xn_ref, hw_ref, dinv_ref, invdeg_ref):
    cs = jnp.maximum(cs_ref[...], 1.0)
    cd = jnp.maximum(cd_ref[...], 1.0)
    xn = 0.5 * (ss_ref[...] / cs + sd_ref[...] / cd)
    deg = wd_ref[...] + 1.0
    dinv_ref[...] = jax.lax.rsqrt(deg)
    invdeg_ref[...] = 1.0 / deg
    xn_ref[...] = xn
    hw_ref[...] = jnp.dot(xn, g1w_ref[...], preferred_element_type=jnp.float32)


def _nodeB_kernel(agg_ref, hw_ref, invdeg_ref, xn_ref,
                  g1b_ref, n1g_ref, n1b_ref, g2w_ref,
                  x1_ref, hw2_ref):
    pre = agg_ref[...] + hw_ref[...] * invdeg_ref[...] + g1b_ref[...]
    t = pre + xn_ref[...]
    z = _ln(t, n1g_ref[...], n1b_ref[...])
    x1 = 0.5 * z * (1.0 + jax.lax.erf(z * 0.7071067811865476))
    x1_ref[...] = x1
    hw2_ref[...] = jnp.dot(x1, g2w_ref[...], preferred_element_type=jnp.float32)


def _nodeC_kernel(agg_ref, hw2_ref, invdeg_ref, x1_ref,
                  g2b_ref, n2g_ref, n2b_ref, batch_ref,
                  ng_sum_ref, ng_cnt_ref):
    i = pl.program_id(0)
    pre = agg_ref[...] + hw2_ref[...] * invdeg_ref[...] + g2b_ref[...]
    xn2 = _ln(pre + x1_ref[...], n2g_ref[...], n2b_ref[...])
    onehot = (batch_ref[...] == jax.lax.broadcasted_iota(jnp.int32, (1, NG), 1)
              ).astype(jnp.float32)
    ng_blk = jax.lax.dot_general(onehot, xn2, (((0,), (0,)), ((), ())),
                                 preferred_element_type=jnp.float32)
    cnt_blk = jax.lax.dot_general(onehot, jnp.ones((NBLK, 1), jnp.float32),
                                  (((0,), (0,)), ((), ())),
                                  preferred_element_type=jnp.float32)

    @pl.when(i == 0)
    def _():
        ng_sum_ref[...] = jnp.zeros_like(ng_sum_ref)
        ng_cnt_ref[...] = jnp.zeros_like(ng_cnt_ref)

    ng_sum_ref[...] += ng_blk
    ng_cnt_ref[...] += cnt_blk


def _head_kernel(ng_sum_ref, ng_cnt_ref, eg_sum_ref, eg_cnt_ref,
                 w1a_ref, w1b_ref, b1_ref, w2_ref, b2_ref, out_ref):
    node_graph = ng_sum_ref[...] / jnp.maximum(ng_cnt_ref[...], 1.0)
    edge_graph = eg_sum_ref[...] / jnp.maximum(eg_cnt_ref[...], 1.0)
    g = jax.nn.relu(
        jnp.dot(node_graph, w1a_ref[...], preferred_element_type=jnp.float32)
        + jnp.dot(edge_graph, w1b_ref[...], preferred_element_type=jnp.float32)
        + b1_ref[...])
    out_ref[...] = (jnp.dot(g, w2_ref[...], preferred_element_type=jnp.float32)
                    + b2_ref[...])


def _full(shape):
    return pl.BlockSpec(shape, lambda *args: tuple(0 for _ in shape))


def kernel(x, edge_attr, array, edge_index, batch, ee_w1, ee_b1, ee_g1, ee_bb1, ee_w2, ee_b2, ee_g2, ee_bb2, ee_w3, ee_b3, ee_g3, ee_bb3, e2n_w, e2n_b, g1_w, g1_b, g2_w, g2_b, n1_g, n1_b, n2_g, n2_b, h_w1, h_b1, h_w2, h_b2):
    src = edge_index[0]
    dst = edge_index[1]

    rel3 = array[src] - array[dst]
    rel = jnp.pad(rel3, ((0, 0), (0, 1)))
    bsrc = batch[src][:, None]

    w1a = ee_w1[:16]
    w1b = jnp.pad(ee_w1[16:19], ((0, 1), (0, 0)))
    w1l = ee_w1[19:20]

    ngrid = E // EBLK
    p, dist, eg_sum, eg_cnt = pl.pallas_call(
        _edge_kernel,
        grid=(ngrid,),
        in_specs=[
            pl.BlockSpec((EBLK, 16), lambda i: (i, 0)),
            pl.BlockSpec((EBLK, 4), lambda i: (i, 0)),
            pl.BlockSpec((EBLK, 1), lambda i: (i, 0)),
            _full((16, 128)), _full((4, 128)), _full((1, 128)),
            _full((1, 128)), _full((1, 128)), _full((1, 128)),
            _full((128, 128)), _full((1, 128)), _full((1, 128)), _full((1, 128)),
            _full((128, 128)), _full((1, 128)), _full((1, 128)), _full((1, 128)),
            _full((128, 128)), _full((1, 128)),
        ],
        out_specs=[
            pl.BlockSpec((EBLK, 128), lambda i: (i, 0)),
            pl.BlockSpec((EBLK, 1), lambda i: (i, 0)),
            _full((NG, 128)),
            _full((NG, 1)),
        ],
        out_shape=[
            jax.ShapeDtypeStruct((E, 128), jnp.float32),
            jax.ShapeDtypeStruct((E, 1), jnp.float32),
            jax.ShapeDtypeStruct((NG, 128), jnp.float32),
            jax.ShapeDtypeStruct((NG, 1), jnp.float32),
        ],
    )(edge_attr, rel, bsrc,
      w1a, w1b, w1l, ee_b1[None], ee_g1[None], ee_bb1[None],
      ee_w2, ee_b2[None], ee_g2[None], ee_bb2[None],
      ee_w3, ee_b3[None], ee_g3[None], ee_bb3[None],
      e2n_w, e2n_b[None])

    distf = dist[:, 0]

    ones = jnp.ones((E,), jnp.float32)
    cnt_src = jnp.zeros((N, 1), jnp.float32).at[src, 0].add(ones)
    cnt_dst = jnp.zeros((N, 1), jnp.float32).at[dst, 0].add(ones)
    sum_src = jnp.zeros((N, 128), jnp.float32).at[src].add(p)
    sum_dst = jnp.zeros((N, 128), jnp.float32).at[dst].add(p)
    wsum_dst = jnp.zeros((N, 1), jnp.float32).at[dst, 0].add(distf)

    ngrid_n = N // NBLK
    nspec = pl.BlockSpec((NBLK, 128), lambda i: (i, 0))
    cspec = pl.BlockSpec((NBLK, 1), lambda i: (i, 0))

    xn, hw, dinv, invdeg = pl.pallas_call(
        _nodeA_kernel,
        grid=(ngrid_n,),
        in_specs=[nspec, cspec, nspec, cspec, cspec, _full((128, 128))],
        out_specs=[nspec, nspec, cspec, cspec],
        out_shape=[
            jax.ShapeDtypeStruct((N, 128), jnp.float32),
            jax.ShapeDtypeStruct((N, 128), jnp.float32),
            jax.ShapeDtypeStruct((N, 1), jnp.float32),
            jax.ShapeDtypeStruct((N, 1), jnp.float32),
        ],
    )(sum_src, cnt_src, sum_dst, cnt_dst, wsum_dst, g1_w)

    dinvf = dinv[:, 0]
    norm = dinvf[src] * distf * dinvf[dst]

    agg1 = jnp.zeros((N, 128), jnp.float32).at[dst].add(_sc_u(hw, src, norm))

    x1, hw2 = pl.pallas_call(
        _nodeB_kernel,
        grid=(ngrid_n,),
        in_specs=[nspec, nspec, cspec, nspec,
                  _full((1, 128)), _full((1, 128)), _full((1, 128)),
                  _full((128, 128))],
        out_specs=[nspec, nspec],
        out_shape=[
            jax.ShapeDtypeStruct((N, 128), jnp.float32),
            jax.ShapeDtypeStruct((N, 128), jnp.float32),
        ],
    )(agg1, hw, invdeg, xn, g1_b[None], n1_g[None], n1_b[None], g2_w)

    agg2 = jnp.zeros((N, 128), jnp.float32).at[dst].add(_sc_u(hw2, src, norm))

    ng_sum, ng_cnt = pl.pallas_call(
        _nodeC_kernel,
        grid=(ngrid_n,),
        in_specs=[nspec, nspec, cspec, nspec,
                  _full((1, 128)), _full((1, 128)), _full((1, 128)),
                  pl.BlockSpec((NBLK, 1), lambda i: (i, 0))],
        out_specs=[_full((NG, 128)), _full((NG, 1))],
        out_shape=[
            jax.ShapeDtypeStruct((NG, 128), jnp.float32),
            jax.ShapeDtypeStruct((NG, 1), jnp.float32),
        ],
    )(agg2, hw2, invdeg, x1, g2_b[None], n2_g[None], n2_b[None],
      batch[:, None])

    w2p = jnp.pad(h_w2, ((0, 0), (0, 126)))
    b2p = jnp.pad(h_b2, (0, 126))[None]
    out128 = pl.pallas_call(
        _head_kernel,
        in_specs=[_full((NG, 128)), _full((NG, 1)),
                  _full((NG, 128)), _full((NG, 1)),
                  _full((128, 128)), _full((128, 128)), _full((1, 128)),
                  _full((128, 128)), _full((1, 128))],
        out_specs=_full((NG, 128)),
        out_shape=jax.ShapeDtypeStruct((NG, 128), jnp.float32),
    )(ng_sum, ng_cnt, eg_sum, eg_cnt,
      h_w1[:128], h_w1[128:], h_b1[None], w2p, b2p)

    return out128[:, :2]


# speedup vs baseline: 1.5638x; 1.4828x over previous
import functools
import jax
import jax.numpy as jnp
from jax import lax
from jax.experimental import pallas as pl
from jax.experimental.pallas import tpu as pltpu
from jax.experimental.pallas import tpu_sc as plsc

N = 50000
E = 800000
NG = 8

EBLK = 1600
NBLK = 5000

SCW = 64
NWIN = E // SCW


def _sc_u_body(h_hbm, src_hbm, scl_hbm, out_hbm, scl_v, sidx, rows, sem):
    c = lax.axis_index("c")
    s = lax.axis_index("s")
    wid = s * 2 + c

    def win(k, _):
        widx = wid + 32 * k

        @pl.when(widx < NWIN)
        def _():
            wb = widx * SCW
            pltpu.sync_copy(scl_hbm.at[pl.ds(wb, SCW)], scl_v)
            pltpu.sync_copy(src_hbm.at[pl.ds(wb, SCW)], sidx)
            pltpu.async_copy(h_hbm.at[sidx], rows, sem).wait()

            def scale(g, _):
                base = g * 16
                nv = scl_v[pl.ds(base, 16)]
                for t in range(16):
                    r = base + t
                    for q in range(8):
                        sl = pl.ds(q * 16, 16)
                        rows[r, sl] = rows[r, sl] * nv[t]
                return 0

            lax.fori_loop(0, SCW // 16, scale, 0)
            pltpu.sync_copy(rows, out_hbm.at[pl.ds(wb, SCW)])
        return 0

    lax.fori_loop(0, (NWIN + 31) // 32, win, 0)


def _sc_mesh():
    return plsc.VectorSubcoreMesh(core_axis_name="c", subcore_axis_name="s")


def _sc_u(h, src, scl):
    return pl.kernel(
        _sc_u_body,
        out_type=jax.ShapeDtypeStruct((E, 128), jnp.float32),
        mesh=_sc_mesh(),
        scratch_types=[
            pltpu.VMEM((SCW,), jnp.float32),
            pltpu.VMEM((SCW,), jnp.int32),
            pltpu.VMEM((SCW, 128), jnp.float32),
            pltpu.SemaphoreType.DMA,
        ],
    )(h, src, scl)


def _ln(h, g, b):
    m = jnp.mean(h, axis=-1, keepdims=True)
    v = jnp.mean((h - m) * (h - m), axis=-1, keepdims=True)
    return (h - m) * jax.lax.rsqrt(v + 1e-5) * g + b


def _edge_kernel(edge_attr_ref, rel_ref, bsrc_ref,
                 w1a_ref, w1b_ref, w1l_ref, b1_ref, g1_ref, bb1_ref,
                 w2_ref, b2_ref, g2_ref, bb2_ref,
                 w3_ref, b3_ref, g3_ref, bb3_ref,
                 e2n_w_ref, e2n_b_ref,
                 p_ref, dist_ref, eg_sum_ref, eg_cnt_ref):
    i = pl.program_id(0)
    rel = rel_ref[...]
    d2 = jnp.sum(rel * rel, axis=-1, keepdims=True)
    dist = jnp.sqrt(d2)
    inv = 1.0 / (dist + 1e-8)
    logd = jnp.log(dist + 1e-6)

    h = (jnp.dot(edge_attr_ref[...], w1a_ref[...],
                 preferred_element_type=jnp.float32)
         + jnp.dot(rel, w1b_ref[...], preferred_element_type=jnp.float32) * inv
         + logd * w1l_ref[...]
         + b1_ref[...])
    h = jax.nn.relu(h)
    h = _ln(h, g1_ref[...], bb1_ref[...])
    h = jax.nn.relu(jnp.dot(h, w2_ref[...], preferred_element_type=jnp.float32)
                    + b2_ref[...])
    h = _ln(h, g2_ref[...], bb2_ref[...])
    h = jax.nn.relu(jnp.dot(h, w3_ref[...], preferred_element_type=jnp.float32)
                    + b3_ref[...])
    emb = _ln(h, g3_ref[...], bb3_ref[...])

    p = jax.nn.relu(jnp.dot(emb, e2n_w_ref[...],
                            preferred_element_type=jnp.float32)
                    + e2n_b_ref[...])
    p_ref[...] = p
    dist_ref[...] = dist

    onehot = (bsrc_ref[...] == jax.lax.broadcasted_iota(jnp.int32, (1, NG), 1)
              ).astype(jnp.float32)
    eg_blk = jax.lax.dot_general(onehot, emb, (((0,), (0,)), ((), ())),
                                 preferred_element_type=jnp.float32)
    cnt_blk = jax.lax.dot_general(onehot, jnp.ones((EBLK, 1), jnp.float32),
                                  (((0,), (0,)), ((), ())),
                                  preferred_element_type=jnp.float32)

    @pl.when(i == 0)
    def _():
        eg_sum_ref[...] = jnp.zeros_like(eg_sum_ref)
        eg_cnt_ref[...] = jnp.zeros_like(eg_cnt_ref)

    eg_sum_ref[...] += eg_blk
    eg_cnt_ref[...] += cnt_blk


def _nodeA_kernel(ss_ref, cs_ref, sd_ref, cd_ref, wd_ref, g1w_ref,
                  xn_ref, hs_ref, dinv_ref):
    cs = jnp.maximum(cs_ref[...], 1.0)
    cd = jnp.maximum(cd_ref[...], 1.0)
    xn = 0.5 * (ss_ref[...] / cs + sd_ref[...] / cd)
    deg = wd_ref[...] + 1.0
    dinv = jax.lax.rsqrt(deg)
    dinv_ref[...] = dinv
    xn_ref[...] = xn
    hs_ref[...] = dinv * jnp.dot(xn, g1w_ref[...],
                                 preferred_element_type=jnp.float32)


def _nodeB_kernel(agg_ref, hs_ref, dinv_ref, xn_ref,
                  g1b_ref, n1g_ref, n1b_ref, g2w_ref,
                  x1_ref, hs2_ref):
    dinv = dinv_ref[...]
    pre = dinv * (agg_ref[...] + hs_ref[...]) + g1b_ref[...]
    t = pre + xn_ref[...]
    z = _ln(t, n1g_ref[...], n1b_ref[...])
    x1 = 0.5 * z * (1.0 + jax.lax.erf(z * 0.7071067811865476))
    x1_ref[...] = x1
    hs2_ref[...] = dinv * jnp.dot(x1, g2w_ref[...],
                                  preferred_element_type=jnp.float32)


def _nodeC_kernel(agg_ref, hs2_ref, dinv_ref, x1_ref,
                  g2b_ref, n2g_ref, n2b_ref, batch_ref,
                  ng_sum_ref, ng_cnt_ref):
    i = pl.program_id(0)
    pre = dinv_ref[...] * (agg_ref[...] + hs2_ref[...]) + g2b_ref[...]
    xn2 = _ln(pre + x1_ref[...], n2g_ref[...], n2b_ref[...])
    onehot = (batch_ref[...] == jax.lax.broadcasted_iota(jnp.int32, (1, NG), 1)
              ).astype(jnp.float32)
    ng_blk = jax.lax.dot_general(onehot, xn2, (((0,), (0,)), ((), ())),
                                 preferred_element_type=jnp.float32)
    cnt_blk = jax.lax.dot_general(onehot, jnp.ones((NBLK, 1), jnp.float32),
                                  (((0,), (0,)), ((), ())),
                                  preferred_element_type=jnp.float32)

    @pl.when(i == 0)
    def _():
        ng_sum_ref[...] = jnp.zeros_like(ng_sum_ref)
        ng_cnt_ref[...] = jnp.zeros_like(ng_cnt_ref)

    ng_sum_ref[...] += ng_blk
    ng_cnt_ref[...] += cnt_blk


def _head_kernel(ng_sum_ref, ng_cnt_ref, eg_sum_ref, eg_cnt_ref,
                 w1a_ref, w1b_ref, b1_ref, w2_ref, b2_ref, out_ref):
    node_graph = ng_sum_ref[...] / jnp.maximum(ng_cnt_ref[...], 1.0)
    edge_graph = eg_sum_ref[...] / jnp.maximum(eg_cnt_ref[...], 1.0)
    g = jax.nn.relu(
        jnp.dot(node_graph, w1a_ref[...], preferred_element_type=jnp.float32)
        + jnp.dot(edge_graph, w1b_ref[...], preferred_element_type=jnp.float32)
        + b1_ref[...])
    out_ref[...] = (jnp.dot(g, w2_ref[...], preferred_element_type=jnp.float32)
                    + b2_ref[...])


def _full(shape):
    return pl.BlockSpec(shape, lambda *args: tuple(0 for _ in shape))


def kernel(x, edge_attr, array, edge_index, batch, ee_w1, ee_b1, ee_g1, ee_bb1, ee_w2, ee_b2, ee_g2, ee_bb2, ee_w3, ee_b3, ee_g3, ee_bb3, e2n_w, e2n_b, g1_w, g1_b, g2_w, g2_b, n1_g, n1_b, n2_g, n2_b, h_w1, h_b1, h_w2, h_b2):
    src = edge_index[0]
    dst = edge_index[1]

    rel3 = array[src] - array[dst]
    rel = jnp.pad(rel3, ((0, 0), (0, 1)))
    bsrc = batch[src][:, None]

    w1a = ee_w1[:16]
    w1b = jnp.pad(ee_w1[16:19], ((0, 1), (0, 0)))
    w1l = ee_w1[19:20]

    ngrid = E // EBLK
    p, dist, eg_sum, eg_cnt = pl.pallas_call(
        _edge_kernel,
        grid=(ngrid,),
        in_specs=[
            pl.BlockSpec((EBLK, 16), lambda i: (i, 0)),
            pl.BlockSpec((EBLK, 4), lambda i: (i, 0)),
            pl.BlockSpec((EBLK, 1), lambda i: (i, 0)),
            _full((16, 128)), _full((4, 128)), _full((1, 128)),
            _full((1, 128)), _full((1, 128)), _full((1, 128)),
            _full((128, 128)), _full((1, 128)), _full((1, 128)), _full((1, 128)),
            _full((128, 128)), _full((1, 128)), _full((1, 128)), _full((1, 128)),
            _full((128, 128)), _full((1, 128)),
        ],
        out_specs=[
            pl.BlockSpec((EBLK, 128), lambda i: (i, 0)),
            pl.BlockSpec((EBLK, 1), lambda i: (i, 0)),
            _full((NG, 128)),
            _full((NG, 1)),
        ],
        out_shape=[
            jax.ShapeDtypeStruct((E, 128), jnp.float32),
            jax.ShapeDtypeStruct((E, 1), jnp.float32),
            jax.ShapeDtypeStruct((NG, 128), jnp.float32),
            jax.ShapeDtypeStruct((NG, 1), jnp.float32),
        ],
    )(edge_attr, rel, bsrc,
      w1a, w1b, w1l, ee_b1[None], ee_g1[None], ee_bb1[None],
      ee_w2, ee_b2[None], ee_g2[None], ee_bb2[None],
      ee_w3, ee_b3[None], ee_g3[None], ee_bb3[None],
      e2n_w, e2n_b[None])

    distf = dist[:, 0]

    ones = jnp.ones((E,), jnp.float32)
    cnt_src = jnp.zeros((N, 1), jnp.float32).at[src, 0].add(ones)
    cnt_dst = jnp.zeros((N, 1), jnp.float32).at[dst, 0].add(ones)
    sum_src = jnp.zeros((N, 128), jnp.float32).at[src].add(p)
    sum_dst = jnp.zeros((N, 128), jnp.float32).at[dst].add(p)
    wsum_dst = jnp.zeros((N, 1), jnp.float32).at[dst, 0].add(distf)

    ngrid_n = N // NBLK
    nspec = pl.BlockSpec((NBLK, 128), lambda i: (i, 0))
    cspec = pl.BlockSpec((NBLK, 1), lambda i: (i, 0))

    xn, hs, dinv = pl.pallas_call(
        _nodeA_kernel,
        grid=(ngrid_n,),
        in_specs=[nspec, cspec, nspec, cspec, cspec, _full((128, 128))],
        out_specs=[nspec, nspec, cspec],
        out_shape=[
            jax.ShapeDtypeStruct((N, 128), jnp.float32),
            jax.ShapeDtypeStruct((N, 128), jnp.float32),
            jax.ShapeDtypeStruct((N, 1), jnp.float32),
        ],
    )(sum_src, cnt_src, sum_dst, cnt_dst, wsum_dst, g1_w)

    agg1 = jnp.zeros((N, 128), jnp.float32).at[dst].add(_sc_u(hs, src, distf))

    x1, hs2 = pl.pallas_call(
        _nodeB_kernel,
        grid=(ngrid_n,),
        in_specs=[nspec, nspec, cspec, nspec,
                  _full((1, 128)), _full((1, 128)), _full((1, 128)),
                  _full((128, 128))],
        out_specs=[nspec, nspec],
        out_shape=[
            jax.ShapeDtypeStruct((N, 128), jnp.float32),
            jax.ShapeDtypeStruct((N, 128), jnp.float32),
        ],
    )(agg1, hs, dinv, xn, g1_b[None], n1_g[None], n1_b[None], g2_w)

    agg2 = jnp.zeros((N, 128), jnp.float32).at[dst].add(_sc_u(hs2, src, distf))

    ng_sum, ng_cnt = pl.pallas_call(
        _nodeC_kernel,
        grid=(ngrid_n,),
        in_specs=[nspec, nspec, cspec, nspec,
                  _full((1, 128)), _full((1, 128)), _full((1, 128)),
                  pl.BlockSpec((NBLK, 1), lambda i: (i, 0))],
        out_specs=[_full((NG, 128)), _full((NG, 1))],
        out_shape=[
            jax.ShapeDtypeStruct((NG, 128), jnp.float32),
            jax.ShapeDtypeStruct((NG, 1), jnp.float32),
        ],
    )(agg2, hs2, dinv, x1, g2_b[None], n2_g[None], n2_b[None],
      batch[:, None])

    w2p = jnp.pad(h_w2, ((0, 0), (0, 126)))
    b2p = jnp.pad(h_b2, (0, 126))[None]
    out128 = pl.pallas_call(
        _head_kernel,
        in_specs=[_full((NG, 128)), _full((NG, 1)),
                  _full((NG, 128)), _full((NG, 1)),
                  _full((128, 128)), _full((128, 128)), _full((1, 128)),
                  _full((128, 128)), _full((1, 128))],
        out_specs=_full((NG, 128)),
        out_shape=jax.ShapeDtypeStruct((NG, 128), jnp.float32),
    )(ng_sum, ng_cnt, eg_sum, eg_cnt,
      h_w1[:128], h_w1[128:], h_b1[None], w2p, b2p)

    return out128[:, :2]
